# baseline (speedup 1.0000x reference)
import functools

import jax
import jax.numpy as jnp
from jax import lax
from jax.experimental import pallas as pl
from jax.experimental.pallas import tpu as pltpu
from jax.experimental.pallas import tpu_sc as plsc

N = 10000
E = 320000
D = 128
ED = 16
H = 128

NPAD = 10240
NB = 1024
EB = 6400

NC = 2
NS = 16
NW = NC * NS
EPW = E // NW
CHA = 80
NCHA = EPW // CHA
CHC = 200
NCHC = EPW // CHC
RPT = NPAD // NS
_MESH = plsc.VectorSubcoreMesh(core_axis_name="c", subcore_axis_name="s")


def _zero_rows(buf, rows, width):
    zero16 = jnp.zeros((16,), jnp.float32)

    def body(i, _):
        for j in range(width // 16):
            buf[i, pl.ds(j * 16, 16)] = zero16
        return _

    lax.fori_loop(0, rows, body, None)


_SC_PARAMS = pltpu.CompilerParams(needs_layout_passes=False)


def _build_sc_aggregate():
    scratch = [
        pltpu.VMEM((EPW,), jnp.int32),
        pltpu.VMEM((NCHA, CHA), jnp.int32),
        pltpu.VMEM((CHA, H), jnp.float32),
        pltpu.VMEM((CHA, H), jnp.float32),
        pltpu.VMEM_SHARED((NPAD, H), jnp.float32),
        pltpu.SemaphoreType.DMA,
        pltpu.SemaphoreType.DMA,
        pltpu.SemaphoreType.DMA,
        pltpu.SemaphoreType.DMA,
    ]

    def body(table, src_hbm, dst3d_hbm, out_agg, src_all, dst_all,
             rows0, rows1, acc, g0, g1, s0, s1):
        c = lax.axis_index("c")
        s = lax.axis_index("s")
        wid = c * NS + s

        pltpu.sync_copy(src_hbm.at[pl.ds(wid * EPW, EPW)], src_all)
        pltpu.sync_copy(dst3d_hbm.at[wid], dst_all)

        _zero_rows(rows0, CHA, H)
        for k in range(RPT // CHA):
            pltpu.sync_copy(rows0, acc.at[pl.ds(s * RPT + k * CHA, CHA)])
        plsc.subcore_barrier()

        def gather(i, rows_v, sem):
            pltpu.async_copy(
                table.at[src_all.at[pl.ds(i * CHA, CHA)]], rows_v, sem)

        def wait_gather(i, rows_v, sem):
            pltpu.make_async_copy(
                table.at[src_all.at[pl.ds(i * CHA, CHA)]], rows_v, sem).wait()

        def scatter(i, rows_v, sem):
            pltpu.async_copy(rows_v, acc.at[dst_all.at[i]], sem, add=True)

        def wait_scatter(i, rows_v, sem):
            pltpu.make_async_copy(rows_v, acc.at[dst_all.at[i]], sem).wait()

        gather(0, rows0, g0)
        gather(1, rows1, g1)

        def pair(k, _):
            i = 2 * k
            wait_gather(i, rows0, g0)
            scatter(i, rows0, s0)
            wait_gather(i + 1, rows1, g1)
            scatter(i + 1, rows1, s1)
            wait_scatter(i, rows0, s0)
            gather(i + 2, rows0, g0)
            wait_scatter(i + 1, rows1, s1)
            gather(i + 3, rows1, g1)
            return _

        lax.fori_loop(0, (NCHA - 3) // 2, pair, None)
        i = NCHA - 3
        wait_gather(i, rows0, g0)
        scatter(i, rows0, s0)
        wait_gather(i + 1, rows1, g1)
        scatter(i + 1, rows1, s1)
        wait_scatter(i, rows0, s0)
        gather(i + 2, rows0, g0)
        wait_gather(i + 2, rows0, g0)
        scatter(i + 2, rows0, s0)
        wait_scatter(i + 1, rows1, s1)
        wait_scatter(i + 2, rows0, s0)
        plsc.subcore_barrier()

        r0 = s * RPT
        for k in range(RPT // CHA):
            pltpu.sync_copy(acc.at[pl.ds(r0 + k * CHA, CHA)], rows0)
            pltpu.sync_copy(rows0, out_agg.at[c, pl.ds(r0 + k * CHA, CHA)])

    return pl.kernel(
        body, out_type=jax.ShapeDtypeStruct((NC, NPAD, H), jnp.float32),
        mesh=_MESH, scratch_types=scratch, compiler_params=_SC_PARAMS)


_sc_aggregate = _build_sc_aggregate()


@functools.partial(
    pl.kernel,
    out_type=jax.ShapeDtypeStruct((NW, NPAD), jnp.float32),
    mesh=_MESH,
    scratch_types=[
        pltpu.VMEM((EPW,), jnp.int32),
        pltpu.VMEM((NPAD,), jnp.float32),
    ],
    compiler_params=_SC_PARAMS,
)
def _sc_degree(dst_hbm, out_deg, dst_all, deg_v):
    c = lax.axis_index("c")
    s = lax.axis_index("s")
    wid = c * NS + s
    pltpu.sync_copy(dst_hbm.at[pl.ds(wid * EPW, EPW)], dst_all)
    zero16 = jnp.zeros((16,), jnp.float32)

    def zdeg(i, _):
        deg_v[pl.ds(i * 16, 16)] = zero16
        return _

    lax.fori_loop(0, NPAD // 16, zdeg, None)
    ones16 = jnp.ones((16,), jnp.float32)

    def count(i, _):
        idx = dst_all[pl.ds(i * 16, 16)]
        plsc.addupdate_scatter(deg_v, [idx], ones16)
        return _

    lax.fori_loop(0, EPW // 16, count, None)
    pltpu.sync_copy(deg_v, out_deg.at[wid])


@functools.partial(
    pl.kernel,
    out_type=jax.ShapeDtypeStruct((E, H // 2), jnp.int32),
    mesh=_MESH,
    scratch_types=[
        pltpu.VMEM((EPW,), jnp.int32),
        pltpu.VMEM((EPW,), jnp.int32),
        pltpu.VMEM((CHC, H // 2), jnp.int32),
        pltpu.VMEM((CHC, H // 2), jnp.int32),
        pltpu.VMEM((CHC, H // 2), jnp.int32),
        pltpu.VMEM((CHC, H // 2), jnp.int32),
        pltpu.SemaphoreType.DMA,
        pltpu.SemaphoreType.DMA,
        pltpu.SemaphoreType.DMA,
        pltpu.SemaphoreType.DMA,
    ],
    compiler_params=pltpu.CompilerParams(needs_layout_passes=False,
                                         use_tc_tiling_on_sc=False),
)
def _sc_edge_combine(ps_hbm, pd_hbm, src_hbm, dst_hbm, out_g,
                     src_all, dst_all, bufa0, bufa1, bufb0, bufb1,
                     g0, g1, w0, w1):
    c = lax.axis_index("c")
    s = lax.axis_index("s")
    wid = c * NS + s
    base = wid * EPW
    pltpu.sync_copy(src_hbm.at[pl.ds(base, EPW)], src_all)
    pltpu.sync_copy(dst_hbm.at[pl.ds(base, EPW)], dst_all)

    def gathers(i, buf_a, buf_b, sem):
        sl = pl.ds(i * CHC, CHC)
        pltpu.async_copy(ps_hbm.at[src_all.at[sl]], buf_a, sem)
        pltpu.async_copy(pd_hbm.at[dst_all.at[sl]], buf_b, sem)

    def wait_gathers(i, buf_a, buf_b, sem):
        sl = pl.ds(i * CHC, CHC)
        pltpu.make_async_copy(ps_hbm.at[src_all.at[sl]], buf_a, sem).wait()
        pltpu.make_async_copy(pd_hbm.at[dst_all.at[sl]], buf_b, sem).wait()

    def add_and_write(i, buf_a, buf_b, sem):
        def add_row(r, _):
            for j in range(H // 32):
                sl = pl.ds(j * 16, 16)
                a = plsc.bitcast(buf_a[r, sl], jnp.bfloat16)
                bv = plsc.bitcast(buf_b[r, sl], jnp.bfloat16)
                buf_a[r, sl] = plsc.bitcast(a + bv, jnp.int32)
            return _

        lax.fori_loop(0, CHC, add_row, None)
        pltpu.async_copy(buf_a, out_g.at[pl.ds(base + i * CHC, CHC)], sem)

    def wait_write(i, buf_a, sem):
        pltpu.make_async_copy(
            buf_a, out_g.at[pl.ds(base + i * CHC, CHC)], sem).wait()

    gathers(0, bufa0, bufb0, g0)
    gathers(1, bufa1, bufb1, g1)

    def pair(k, _):
        i = 2 * k
        wait_gathers(i, bufa0, bufb0, g0)
        add_and_write(i, bufa0, bufb0, w0)
        wait_gathers(i + 1, bufa1, bufb1, g1)
        add_and_write(i + 1, bufa1, bufb1, w1)
        wait_write(i, bufa0, w0)
        gathers(i + 2, bufa0, bufb0, g0)
        wait_write(i + 1, bufa1, w1)
        gathers(i + 3, bufa1, bufb1, g1)
        return _

    lax.fori_loop(0, NCHC // 2 - 1, pair, None)
    i = NCHC - 2
    wait_gathers(i, bufa0, bufb0, g0)
    add_and_write(i, bufa0, bufb0, w0)
    wait_gathers(i + 1, bufa1, bufb1, g1)
    add_and_write(i + 1, bufa1, bufb1, w1)
    wait_write(i, bufa0, w0)
    wait_write(i + 1, bufa1, w1)


def _tc_two_matmul(x, wa, wb):
    def body(x_ref, wa_ref, wb_ref, oa_ref, ob_ref):
        xv = x_ref[...]
        oa_ref[...] = jnp.dot(xv, wa_ref[...], preferred_element_type=jnp.float32)
        ob_ref[...] = jnp.dot(xv, wb_ref[...], preferred_element_type=jnp.float32)

    return pl.pallas_call(
        body,
        grid=(NPAD // NB,),
        in_specs=[
            pl.BlockSpec((NB, D), lambda i: (i, 0)),
            pl.BlockSpec((D, H), lambda i: (0, 0)),
            pl.BlockSpec((D, H), lambda i: (0, 0)),
        ],
        out_specs=[
            pl.BlockSpec((NB, H), lambda i: (i, 0)),
            pl.BlockSpec((NB, H), lambda i: (i, 0)),
        ],
        out_shape=[
            jax.ShapeDtypeStruct((NPAD, H), jnp.float32),
            jax.ShapeDtypeStruct((NPAD, H), jnp.float32),
        ],
    )(x, wa, wb)


def _tc_sage_update(xs, parts, degp, b, wa, wb, act: bool, pack: bool = False):
    def body(xs_ref, p_ref, dp_ref, b_ref, wa_ref, wb_ref, oa_ref, ob_ref):
        deg = jnp.sum(dp_ref[...], axis=0)
        rdeg = 1.0 / jnp.maximum(deg, 1.0)
        psum = p_ref[0] + p_ref[1]
        h = xs_ref[...] + psum * rdeg[:, None] + b_ref[...]
        if act:
            h = jnp.maximum(h, 0.0)
        ma = jnp.dot(h, wa_ref[...], preferred_element_type=jnp.float32)
        mb = jnp.dot(h, wb_ref[...], preferred_element_type=jnp.float32)
        if pack:
            ua = lax.bitcast_convert_type(ma, jnp.uint32) + jnp.uint32(0x8000)
            ub = lax.bitcast_convert_type(mb, jnp.uint32) + jnp.uint32(0x8000)
            oa_ref[...] = lax.bitcast_convert_type(
                (ua[:, :64] >> 16) | (ua[:, 64:] & jnp.uint32(0xFFFF0000)),
                jnp.int32)
            ob_ref[...] = lax.bitcast_convert_type(
                (ub[:, :64] >> 16) | (ub[:, 64:] & jnp.uint32(0xFFFF0000)),
                jnp.int32)
        else:
            oa_ref[...] = ma
            ob_ref[...] = mb

    return pl.pallas_call(
        body,
        grid=(NPAD // NB,),
        in_specs=[
            pl.BlockSpec((NB, H), lambda i: (i, 0)),
            pl.BlockSpec((NC, NB, H), lambda i: (0, i, 0)),
            pl.BlockSpec((NW, NB), lambda i: (0, i)),
            pl.BlockSpec((1, H), lambda i: (0, 0)),
            pl.BlockSpec((H, H), lambda i: (0, 0)),
            pl.BlockSpec((H, H), lambda i: (0, 0)),
        ],
        out_specs=[
            pl.BlockSpec((NB, H // 2 if pack else H), lambda i: (i, 0)),
            pl.BlockSpec((NB, H // 2 if pack else H), lambda i: (i, 0)),
        ],
        out_shape=[
            jax.ShapeDtypeStruct((NPAD, H // 2 if pack else H),
                                 jnp.int32 if pack else jnp.float32),
            jax.ShapeDtypeStruct((NPAD, H // 2 if pack else H),
                                 jnp.int32 if pack else jnp.float32),
        ],
    )(xs, parts, degp, b, wa, wb)


def _tc_edge_mlp(g, ef, wc, bm1, wm2, bm2, wm3, bm3):
    def body(g_ref, ef_ref, wc_ref, b1_ref, w2_ref, b2_ref, w3_ref, b3_ref,
             out_ref):
        gu = lax.bitcast_convert_type(g_ref[...], jnp.uint32)
        glo = lax.bitcast_convert_type(gu << 16, jnp.float32)
        ghi = lax.bitcast_convert_type(gu & jnp.uint32(0xFFFF0000),
                                       jnp.float32)
        gf = jnp.concatenate([glo, ghi], axis=1)
        z1 = gf + jnp.dot(ef_ref[...], wc_ref[...],
                          preferred_element_type=jnp.float32)
        z1 = jnp.maximum(z1 + b1_ref[...], 0.0)
        z2 = jnp.dot(z1, w2_ref[...], preferred_element_type=jnp.float32)
        z2 = jnp.maximum(z2 + b2_ref[...], 0.0)
        out_ref[...] = jnp.dot(z2, w3_ref[...],
                               preferred_element_type=jnp.float32) + b3_ref[...]

    return pl.pallas_call(
        body,
        grid=(E // EB,),
        in_specs=[
            pl.BlockSpec((EB, H // 2), lambda i: (i, 0)),
            pl.BlockSpec((EB, ED), lambda i: (i, 0)),
            pl.BlockSpec((ED, H), lambda i: (0, 0)),
            pl.BlockSpec((1, H), lambda i: (0, 0)),
            pl.BlockSpec((H, H // 2), lambda i: (0, 0)),
            pl.BlockSpec((1, H // 2), lambda i: (0, 0)),
            pl.BlockSpec((H // 2, 1), lambda i: (0, 0)),
            pl.BlockSpec((1, 1), lambda i: (0, 0)),
        ],
        out_specs=pl.BlockSpec((EB, 1), lambda i: (i, 0)),
        out_shape=jax.ShapeDtypeStruct((E, 1), jnp.float32),
    )(g, ef, wc, bm1, wm2, bm2, wm3, bm3)


def kernel(node_features, edge_index, edge_features,
           W_self1, W_neigh1, b1, W_self2, W_neigh2, b2,
           Wm1, bm1, Wm2, bm2, Wm3, bm3):
    src = edge_index[0]
    dst = edge_index[1]
    dst3d = dst.reshape(NW, NCHA, CHA)
    x = jnp.concatenate(
        [node_features, jnp.zeros((NPAD - N, D), jnp.float32)], axis=0)

    degp = _sc_degree(dst)

    xs1, xw1 = _tc_two_matmul(x, W_self1, W_neigh1)
    p1 = _sc_aggregate(xw1, src, dst3d)
    hs2, hw2 = _tc_sage_update(xs1, p1, degp, b1.reshape(1, H),
                               W_self2, W_neigh2, act=True)

    p2 = _sc_aggregate(hw2, src, dst3d)
    ps, pd = _tc_sage_update(hs2, p2, degp, b2.reshape(1, H),
                             Wm1[:H], Wm1[H:2 * H], act=False, pack=True)

    g = _sc_edge_combine(ps, pd, src, dst)
    logits = _tc_edge_mlp(g, edge_features, Wm1[2 * H:], bm1.reshape(1, H),
                          Wm2, bm2.reshape(1, H // 2), Wm3,
                          bm3.reshape(1, 1))
    return logits.reshape(E)


# speedup vs baseline: 1.1535x; 1.1535x over previous
import functools

import jax
import jax.numpy as jnp
from jax import lax
from jax.experimental import pallas as pl
from jax.experimental.pallas import tpu as pltpu
from jax.experimental.pallas import tpu_sc as plsc

N = 10000
E = 320000
D = 128
ED = 16
H = 128

NPAD = 10240
NB = 1024
EB = 6400

NC = 2
NS = 16
NW = NC * NS
EPW = E // NW
CHA = 80
NCHA = EPW // CHA
CHC = 200
NCHC = EPW // CHC
RPT = NPAD // NS
_MESH = plsc.VectorSubcoreMesh(core_axis_name="c", subcore_axis_name="s")


def _zero_rows(buf, rows, width):
    zero16 = jnp.zeros((16,), jnp.float32)

    def body(i, _):
        for j in range(width // 16):
            buf[i, pl.ds(j * 16, 16)] = zero16
        return _

    lax.fori_loop(0, rows, body, None)


_SC_PARAMS = pltpu.CompilerParams(needs_layout_passes=False)


def _build_sc_aggregate():
    scratch = [
        pltpu.VMEM((EPW,), jnp.int32),
        pltpu.VMEM((NCHA, CHA), jnp.int32),
        pltpu.VMEM((CHA, H), jnp.float32),
        pltpu.VMEM((CHA, H), jnp.float32),
        pltpu.VMEM_SHARED((NPAD, H), jnp.float32),
        pltpu.SemaphoreType.DMA,
        pltpu.SemaphoreType.DMA,
        pltpu.SemaphoreType.DMA,
        pltpu.SemaphoreType.DMA,
    ]

    def body(table, src_hbm, dst3d_hbm, out_agg, src_all, dst_all,
             rows0, rows1, acc, g0, g1, s0, s1):
        c = lax.axis_index("c")
        s = lax.axis_index("s")
        wid = c * NS + s

        pltpu.sync_copy(src_hbm.at[pl.ds(wid * EPW, EPW)], src_all)
        pltpu.sync_copy(dst3d_hbm.at[wid], dst_all)

        _zero_rows(rows0, CHA, H)
        for k in range(RPT // CHA):
            pltpu.sync_copy(rows0, acc.at[pl.ds(s * RPT + k * CHA, CHA)])
        plsc.subcore_barrier()

        def gather(i, rows_v, sem):
            pltpu.async_copy(
                table.at[src_all.at[pl.ds(i * CHA, CHA)]], rows_v, sem)

        def wait_gather(i, rows_v, sem):
            pltpu.make_async_copy(
                table.at[src_all.at[pl.ds(i * CHA, CHA)]], rows_v, sem).wait()

        def scatter(i, rows_v, sem):
            pltpu.async_copy(rows_v, acc.at[dst_all.at[i]], sem, add=True)

        def wait_scatter(i, rows_v, sem):
            pltpu.make_async_copy(rows_v, acc.at[dst_all.at[i]], sem).wait()

        gather(0, rows0, g0)
        gather(1, rows1, g1)

        def pair(k, _):
            i = 2 * k
            wait_gather(i, rows0, g0)
            scatter(i, rows0, s0)
            wait_gather(i + 1, rows1, g1)
            scatter(i + 1, rows1, s1)
            wait_scatter(i, rows0, s0)
            gather(i + 2, rows0, g0)
            wait_scatter(i + 1, rows1, s1)
            gather(i + 3, rows1, g1)
            return _

        lax.fori_loop(0, (NCHA - 3) // 2, pair, None)
        i = NCHA - 3
        wait_gather(i, rows0, g0)
        scatter(i, rows0, s0)
        wait_gather(i + 1, rows1, g1)
        scatter(i + 1, rows1, s1)
        wait_scatter(i, rows0, s0)
        gather(i + 2, rows0, g0)
        wait_gather(i + 2, rows0, g0)
        scatter(i + 2, rows0, s0)
        wait_scatter(i + 1, rows1, s1)
        wait_scatter(i + 2, rows0, s0)
        plsc.subcore_barrier()

        r0 = s * RPT
        for k in range(RPT // CHA):
            pltpu.sync_copy(acc.at[pl.ds(r0 + k * CHA, CHA)], rows0)
            pltpu.sync_copy(rows0, out_agg.at[c, pl.ds(r0 + k * CHA, CHA)])

    return pl.kernel(
        body, out_type=jax.ShapeDtypeStruct((NC, NPAD, H), jnp.float32),
        mesh=_MESH, scratch_types=scratch, compiler_params=_SC_PARAMS)


_sc_aggregate = _build_sc_aggregate()


@functools.partial(
    pl.kernel,
    out_type=jax.ShapeDtypeStruct((NW, NPAD), jnp.float32),
    mesh=_MESH,
    scratch_types=[
        pltpu.VMEM((EPW,), jnp.int32),
        pltpu.VMEM((NPAD,), jnp.float32),
    ],
    compiler_params=_SC_PARAMS,
)
def _sc_degree(dst_hbm, out_deg, dst_all, deg_v):
    c = lax.axis_index("c")
    s = lax.axis_index("s")
    wid = c * NS + s
    pltpu.sync_copy(dst_hbm.at[pl.ds(wid * EPW, EPW)], dst_all)
    zero16 = jnp.zeros((16,), jnp.float32)

    def zdeg(i, _):
        deg_v[pl.ds(i * 16, 16)] = zero16
        return _

    lax.fori_loop(0, NPAD // 16, zdeg, None)
    ones16 = jnp.ones((16,), jnp.float32)

    def count(i, _):
        idx = dst_all[pl.ds(i * 16, 16)]
        plsc.addupdate_scatter(deg_v, [idx], ones16)
        return _

    lax.fori_loop(0, EPW // 16, count, None)
    pltpu.sync_copy(deg_v, out_deg.at[wid])


@functools.partial(
    pl.kernel,
    out_type=jax.ShapeDtypeStruct((E // 2, H), jnp.int32),
    mesh=_MESH,
    scratch_types=[
        pltpu.VMEM((EPW,), jnp.int32),
        pltpu.VMEM((EPW,), jnp.int32),
        pltpu.VMEM((CHC, H // 2), jnp.int32),
        pltpu.VMEM((CHC, H // 2), jnp.int32),
        pltpu.VMEM((CHC, H // 2), jnp.int32),
        pltpu.VMEM((CHC, H // 2), jnp.int32),
        pltpu.VMEM((CHC // 2, H), jnp.int32),
        pltpu.VMEM((CHC // 2, H), jnp.int32),
        pltpu.SemaphoreType.DMA,
        pltpu.SemaphoreType.DMA,
        pltpu.SemaphoreType.DMA,
        pltpu.SemaphoreType.DMA,
    ],
    compiler_params=pltpu.CompilerParams(needs_layout_passes=False,
                                         use_tc_tiling_on_sc=False),
)
def _sc_edge_combine(ps_hbm, pd_hbm, src_hbm, dst_hbm, out_g,
                     src_all, dst_all, bufa0, bufa1, bufb0, bufb1,
                     wb0, wb1, g0, g1, w0, w1):
    c = lax.axis_index("c")
    s = lax.axis_index("s")
    wid = c * NS + s
    base = wid * EPW
    pltpu.sync_copy(src_hbm.at[pl.ds(base, EPW)], src_all)
    pltpu.sync_copy(dst_hbm.at[pl.ds(base, EPW)], dst_all)

    def gathers(i, buf_a, buf_b, sem):
        sl = pl.ds(i * CHC, CHC)
        pltpu.async_copy(ps_hbm.at[src_all.at[sl]], buf_a, sem)
        pltpu.async_copy(pd_hbm.at[dst_all.at[sl]], buf_b, sem)

    def wait_gathers(i, buf_a, buf_b, sem):
        sl = pl.ds(i * CHC, CHC)
        pltpu.make_async_copy(ps_hbm.at[src_all.at[sl]], buf_a, sem).wait()
        pltpu.make_async_copy(pd_hbm.at[dst_all.at[sl]], buf_b, sem).wait()

    def add_and_write(i, buf_a, buf_b, wbuf, sem):
        def add_row(r2, _):
            for e in range(2):
                for j in range(H // 32):
                    sl = pl.ds(j * 16, 16)
                    a = plsc.bitcast(buf_a[2 * r2 + e, sl], jnp.bfloat16)
                    bv = plsc.bitcast(buf_b[2 * r2 + e, sl], jnp.bfloat16)
                    wbuf[r2, pl.ds(e * 64 + j * 16, 16)] = plsc.bitcast(
                        a + bv, jnp.int32)
            return _

        lax.fori_loop(0, CHC // 2, add_row, None)
        pltpu.async_copy(
            wbuf, out_g.at[pl.ds((base + i * CHC) // 2, CHC // 2)], sem)

    def wait_write(i, wbuf, sem):
        pltpu.make_async_copy(
            wbuf, out_g.at[pl.ds((base + i * CHC) // 2, CHC // 2)],
            sem).wait()

    gathers(0, bufa0, bufb0, g0)
    gathers(1, bufa1, bufb1, g1)

    def pair(k, _):
        i = 2 * k
        wait_gathers(i, bufa0, bufb0, g0)
        add_and_write(i, bufa0, bufb0, wb0, w0)
        wait_gathers(i + 1, bufa1, bufb1, g1)
        add_and_write(i + 1, bufa1, bufb1, wb1, w1)
        wait_write(i, wb0, w0)
        gathers(i + 2, bufa0, bufb0, g0)
        wait_write(i + 1, wb1, w1)
        gathers(i + 3, bufa1, bufb1, g1)
        return _

    lax.fori_loop(0, NCHC // 2 - 1, pair, None)
    i = NCHC - 2
    wait_gathers(i, bufa0, bufb0, g0)
    add_and_write(i, bufa0, bufb0, wb0, w0)
    wait_gathers(i + 1, bufa1, bufb1, g1)
    add_and_write(i + 1, bufa1, bufb1, wb1, w1)
    wait_write(i, wb0, w0)
    wait_write(i + 1, wb1, w1)


def _tc_two_matmul(x, wa, wb):
    def body(x_ref, wa_ref, wb_ref, oa_ref, ob_ref):
        xv = x_ref[...]
        oa_ref[...] = jnp.dot(xv, wa_ref[...], preferred_element_type=jnp.float32)
        ob_ref[...] = jnp.dot(xv, wb_ref[...], preferred_element_type=jnp.float32)

    return pl.pallas_call(
        body,
        grid=(NPAD // NB,),
        in_specs=[
            pl.BlockSpec((NB, D), lambda i: (i, 0)),
            pl.BlockSpec((D, H), lambda i: (0, 0)),
            pl.BlockSpec((D, H), lambda i: (0, 0)),
        ],
        out_specs=[
            pl.BlockSpec((NB, H), lambda i: (i, 0)),
            pl.BlockSpec((NB, H), lambda i: (i, 0)),
        ],
        out_shape=[
            jax.ShapeDtypeStruct((NPAD, H), jnp.float32),
            jax.ShapeDtypeStruct((NPAD, H), jnp.float32),
        ],
    )(x, wa, wb)


def _tc_sage_update(xs, parts, degp, b, wa, wb, act: bool, pack: bool = False):
    def body(xs_ref, p_ref, dp_ref, b_ref, wa_ref, wb_ref, oa_ref, ob_ref):
        deg = jnp.sum(dp_ref[...], axis=0)
        rdeg = 1.0 / jnp.maximum(deg, 1.0)
        psum = p_ref[0] + p_ref[1]
        h = xs_ref[...] + psum * rdeg[:, None] + b_ref[...]
        if act:
            h = jnp.maximum(h, 0.0)
        ma = jnp.dot(h, wa_ref[...], preferred_element_type=jnp.float32)
        mb = jnp.dot(h, wb_ref[...], preferred_element_type=jnp.float32)
        if pack:
            ua = lax.bitcast_convert_type(ma, jnp.uint32) + jnp.uint32(0x8000)
            ub = lax.bitcast_convert_type(mb, jnp.uint32) + jnp.uint32(0x8000)
            oa_ref[...] = lax.bitcast_convert_type(
                (ua[:, :64] >> 16) | (ua[:, 64:] & jnp.uint32(0xFFFF0000)),
                jnp.int32)
            ob_ref[...] = lax.bitcast_convert_type(
                (ub[:, :64] >> 16) | (ub[:, 64:] & jnp.uint32(0xFFFF0000)),
                jnp.int32)
        else:
            oa_ref[...] = ma
            ob_ref[...] = mb

    return pl.pallas_call(
        body,
        grid=(NPAD // NB,),
        in_specs=[
            pl.BlockSpec((NB, H), lambda i: (i, 0)),
            pl.BlockSpec((NC, NB, H), lambda i: (0, i, 0)),
            pl.BlockSpec((NW, NB), lambda i: (0, i)),
            pl.BlockSpec((1, H), lambda i: (0, 0)),
            pl.BlockSpec((H, H), lambda i: (0, 0)),
            pl.BlockSpec((H, H), lambda i: (0, 0)),
        ],
        out_specs=[
            pl.BlockSpec((NB, H // 2 if pack else H), lambda i: (i, 0)),
            pl.BlockSpec((NB, H // 2 if pack else H), lambda i: (i, 0)),
        ],
        out_shape=[
            jax.ShapeDtypeStruct((NPAD, H // 2 if pack else H),
                                 jnp.int32 if pack else jnp.float32),
            jax.ShapeDtypeStruct((NPAD, H // 2 if pack else H),
                                 jnp.int32 if pack else jnp.float32),
        ],
    )(xs, parts, degp, b, wa, wb)


def _tc_edge_mlp(g, efp, wc, bm1, wm2, bm2, wm3, bm3):
    def body(g_ref, ef_ref, wc_ref, b1_ref, w2_ref, b2_ref, w3_ref, b3_ref,
             out_ref):
        gu = lax.bitcast_convert_type(g_ref[...], jnp.uint32)
        outs = []
        for e in range(2):
            ge = gu[:, e * 64:(e + 1) * 64]
            glo = lax.bitcast_convert_type(ge << 16, jnp.float32)
            ghi = lax.bitcast_convert_type(ge & jnp.uint32(0xFFFF0000),
                                           jnp.float32)
            gf = jnp.concatenate([glo, ghi], axis=1)
            efe = ef_ref[:, e * ED:(e + 1) * ED]
            z1 = gf + jnp.dot(efe, wc_ref[...],
                              preferred_element_type=jnp.float32)
            z1 = jnp.maximum(z1 + b1_ref[...], 0.0)
            z2 = jnp.dot(z1, w2_ref[...], preferred_element_type=jnp.float32)
            z2 = jnp.maximum(z2 + b2_ref[...], 0.0)
            outs.append(jnp.dot(z2, w3_ref[...],
                                preferred_element_type=jnp.float32)
                        + b3_ref[...])
        out_ref[...] = jnp.concatenate(outs, axis=1)

    return pl.pallas_call(
        body,
        grid=(E // EB,),
        in_specs=[
            pl.BlockSpec((EB // 2, H), lambda i: (i, 0)),
            pl.BlockSpec((EB // 2, 2 * ED), lambda i: (i, 0)),
            pl.BlockSpec((ED, H), lambda i: (0, 0)),
            pl.BlockSpec((1, H), lambda i: (0, 0)),
            pl.BlockSpec((H, H // 2), lambda i: (0, 0)),
            pl.BlockSpec((1, H // 2), lambda i: (0, 0)),
            pl.BlockSpec((H // 2, 1), lambda i: (0, 0)),
            pl.BlockSpec((1, 1), lambda i: (0, 0)),
        ],
        out_specs=pl.BlockSpec((EB // 2, 2), lambda i: (i, 0)),
        out_shape=jax.ShapeDtypeStruct((E // 2, 2), jnp.float32),
    )(g, efp, wc, bm1, wm2, bm2, wm3, bm3)


def kernel(node_features, edge_index, edge_features,
           W_self1, W_neigh1, b1, W_self2, W_neigh2, b2,
           Wm1, bm1, Wm2, bm2, Wm3, bm3):
    src = edge_index[0]
    dst = edge_index[1]
    dst3d = dst.reshape(NW, NCHA, CHA)
    x = jnp.concatenate(
        [node_features, jnp.zeros((NPAD - N, D), jnp.float32)], axis=0)

    degp = _sc_degree(dst)

    xs1, xw1 = _tc_two_matmul(x, W_self1, W_neigh1)
    p1 = _sc_aggregate(xw1, src, dst3d)
    hs2, hw2 = _tc_sage_update(xs1, p1, degp, b1.reshape(1, H),
                               W_self2, W_neigh2, act=True)

    p2 = _sc_aggregate(hw2, src, dst3d)
    ps, pd = _tc_sage_update(hs2, p2, degp, b2.reshape(1, H),
                             Wm1[:H], Wm1[H:2 * H], act=False, pack=True)

    g = _sc_edge_combine(ps, pd, src, dst)
    logits = _tc_edge_mlp(g, edge_features.reshape(E // 2, 2 * ED),
                          Wm1[2 * H:], bm1.reshape(1, H),
                          Wm2, bm2.reshape(1, H // 2), Wm3,
                          bm3.reshape(1, 1))
    return logits.reshape(E)


# speedup vs baseline: 1.1643x; 1.0094x over previous
import functools

import jax
import jax.numpy as jnp
from jax import lax
from jax.experimental import pallas as pl
from jax.experimental.pallas import tpu as pltpu
from jax.experimental.pallas import tpu_sc as plsc

N = 10000
E = 320000
D = 128
ED = 16
H = 128

NPAD = 10240
NB = 1024
EB = 6400

NC = 2
NS = 16
NW = NC * NS
EPW = E // NW
CHA = 80
NCHA = EPW // CHA
CHC = 200
NCHC = EPW // CHC
RPT = NPAD // NS
_MESH = plsc.VectorSubcoreMesh(core_axis_name="c", subcore_axis_name="s")


def _zero_rows(buf, rows, width):
    zero16 = jnp.zeros((16,), jnp.float32)

    def body(i, _):
        for j in range(width // 16):
            buf[i, pl.ds(j * 16, 16)] = zero16
        return _

    lax.fori_loop(0, rows, body, None)


_SC_PARAMS = pltpu.CompilerParams(needs_layout_passes=False)


def _build_sc_aggregate():
    scratch = [
        pltpu.VMEM((EPW,), jnp.int32),
        pltpu.VMEM((NCHA, CHA), jnp.int32),
        pltpu.VMEM((CHA, H), jnp.float32),
        pltpu.VMEM((CHA, H), jnp.float32),
        pltpu.VMEM_SHARED((NPAD, H), jnp.float32),
        pltpu.SemaphoreType.DMA,
        pltpu.SemaphoreType.DMA,
        pltpu.SemaphoreType.DMA,
        pltpu.SemaphoreType.DMA,
    ]

    def body(table, src_hbm, dst3d_hbm, out_agg, src_all, dst_all,
             rows0, rows1, acc, g0, g1, s0, s1):
        c = lax.axis_index("c")
        s = lax.axis_index("s")
        wid = c * NS + s

        pltpu.sync_copy(src_hbm.at[pl.ds(wid * EPW, EPW)], src_all)
        pltpu.sync_copy(dst3d_hbm.at[wid], dst_all)

        _zero_rows(rows0, CHA, H)
        for k in range(RPT // CHA):
            pltpu.sync_copy(rows0, acc.at[pl.ds(s * RPT + k * CHA, CHA)])
        plsc.subcore_barrier()

        def gather(i, rows_v, sem):
            pltpu.async_copy(
                table.at[src_all.at[pl.ds(i * CHA, CHA)]], rows_v, sem)

        def wait_gather(i, rows_v, sem):
            pltpu.make_async_copy(
                table.at[src_all.at[pl.ds(i * CHA, CHA)]], rows_v, sem).wait()

        def scatter(i, rows_v, sem):
            pltpu.async_copy(rows_v, acc.at[dst_all.at[i]], sem, add=True)

        def wait_scatter(i, rows_v, sem):
            pltpu.make_async_copy(rows_v, acc.at[dst_all.at[i]], sem).wait()

        gather(0, rows0, g0)
        gather(1, rows1, g1)

        def pair(k, _):
            i = 2 * k
            wait_gather(i, rows0, g0)
            scatter(i, rows0, s0)
            wait_gather(i + 1, rows1, g1)
            scatter(i + 1, rows1, s1)
            wait_scatter(i, rows0, s0)
            gather(i + 2, rows0, g0)
            wait_scatter(i + 1, rows1, s1)
            gather(i + 3, rows1, g1)
            return _

        lax.fori_loop(0, (NCHA - 3) // 2, pair, None)
        i = NCHA - 3
        wait_gather(i, rows0, g0)
        scatter(i, rows0, s0)
        wait_gather(i + 1, rows1, g1)
        scatter(i + 1, rows1, s1)
        wait_scatter(i, rows0, s0)
        gather(i + 2, rows0, g0)
        wait_gather(i + 2, rows0, g0)
        scatter(i + 2, rows0, s0)
        wait_scatter(i + 1, rows1, s1)
        wait_scatter(i + 2, rows0, s0)
        plsc.subcore_barrier()

        r0 = s * RPT
        for k in range(RPT // CHA):
            pltpu.sync_copy(acc.at[pl.ds(r0 + k * CHA, CHA)], rows0)
            pltpu.sync_copy(rows0, out_agg.at[c, pl.ds(r0 + k * CHA, CHA)])

    return pl.kernel(
        body, out_type=jax.ShapeDtypeStruct((NC, NPAD, H), jnp.float32),
        mesh=_MESH, scratch_types=scratch, compiler_params=_SC_PARAMS)


_sc_aggregate = _build_sc_aggregate()


@functools.partial(
    pl.kernel,
    out_type=jax.ShapeDtypeStruct((NW, NPAD), jnp.float32),
    mesh=_MESH,
    scratch_types=[
        pltpu.VMEM((EPW,), jnp.int32),
        pltpu.VMEM((NPAD,), jnp.float32),
    ],
    compiler_params=_SC_PARAMS,
)
def _sc_degree(dst_hbm, out_deg, dst_all, deg_v):
    c = lax.axis_index("c")
    s = lax.axis_index("s")
    wid = c * NS + s
    pltpu.sync_copy(dst_hbm.at[pl.ds(wid * EPW, EPW)], dst_all)
    zero16 = jnp.zeros((16,), jnp.float32)

    def zdeg(i, _):
        deg_v[pl.ds(i * 16, 16)] = zero16
        return _

    lax.fori_loop(0, NPAD // 16, zdeg, None)
    ones16 = jnp.ones((16,), jnp.float32)

    def count(i, _):
        idx = dst_all[pl.ds(i * 16, 16)]
        plsc.addupdate_scatter(deg_v, [idx], ones16)
        return _

    lax.fori_loop(0, EPW // 16, count, None)
    pltpu.sync_copy(deg_v, out_deg.at[wid])


@functools.partial(
    pl.kernel,
    out_type=jax.ShapeDtypeStruct((E // 2, H), jnp.int32),
    mesh=_MESH,
    scratch_types=[
        pltpu.VMEM((EPW,), jnp.int32),
        pltpu.VMEM((EPW,), jnp.int32),
        pltpu.VMEM((CHC, H // 2), jnp.int32),
        pltpu.VMEM((CHC, H // 2), jnp.int32),
        pltpu.VMEM((CHC, H // 2), jnp.int32),
        pltpu.VMEM((CHC, H // 2), jnp.int32),
        pltpu.VMEM((CHC // 2, H), jnp.int32),
        pltpu.VMEM((CHC // 2, H), jnp.int32),
        pltpu.SemaphoreType.DMA,
        pltpu.SemaphoreType.DMA,
        pltpu.SemaphoreType.DMA,
        pltpu.SemaphoreType.DMA,
    ],
    compiler_params=pltpu.CompilerParams(needs_layout_passes=False,
                                         use_tc_tiling_on_sc=False),
)
def _sc_edge_combine(ps_hbm, pd_hbm, src_hbm, dst_hbm, out_g,
                     src_all, dst_all, bufa0, bufa1, bufb0, bufb1,
                     wb0, wb1, g0, g1, w0, w1):
    c = lax.axis_index("c")
    s = lax.axis_index("s")
    wid = c * NS + s
    base = wid * EPW
    pltpu.sync_copy(src_hbm.at[pl.ds(base, EPW)], src_all)
    pltpu.sync_copy(dst_hbm.at[pl.ds(base, EPW)], dst_all)

    def gathers(i, buf_a, buf_b, sem):
        sl = pl.ds(i * CHC, CHC)
        pltpu.async_copy(ps_hbm.at[src_all.at[sl]], buf_a, sem)
        pltpu.async_copy(pd_hbm.at[dst_all.at[sl]], buf_b, sem)

    def wait_gathers(i, buf_a, buf_b, sem):
        sl = pl.ds(i * CHC, CHC)
        pltpu.make_async_copy(ps_hbm.at[src_all.at[sl]], buf_a, sem).wait()
        pltpu.make_async_copy(pd_hbm.at[dst_all.at[sl]], buf_b, sem).wait()

    def add_and_write(i, buf_a, buf_b, wbuf, sem):
        def add_row(r2, _):
            for e in range(2):
                for j in range(H // 32):
                    sl = pl.ds(j * 16, 16)
                    a = plsc.bitcast(buf_a[2 * r2 + e, sl], jnp.bfloat16)
                    bv = plsc.bitcast(buf_b[2 * r2 + e, sl], jnp.bfloat16)
                    wbuf[r2, pl.ds(e * 64 + j * 16, 16)] = plsc.bitcast(
                        a + bv, jnp.int32)
            return _

        lax.fori_loop(0, CHC // 2, add_row, None)
        pltpu.async_copy(
            wbuf, out_g.at[pl.ds((base + i * CHC) // 2, CHC // 2)], sem)

    def wait_write(i, wbuf, sem):
        pltpu.make_async_copy(
            wbuf, out_g.at[pl.ds((base + i * CHC) // 2, CHC // 2)],
            sem).wait()

    gathers(0, bufa0, bufb0, g0)
    gathers(1, bufa1, bufb1, g1)

    def pair(k, _):
        i = 2 * k
        wait_gathers(i, bufa0, bufb0, g0)
        add_and_write(i, bufa0, bufb0, wb0, w0)
        wait_gathers(i + 1, bufa1, bufb1, g1)
        add_and_write(i + 1, bufa1, bufb1, wb1, w1)
        wait_write(i, wb0, w0)
        gathers(i + 2, bufa0, bufb0, g0)
        wait_write(i + 1, wb1, w1)
        gathers(i + 3, bufa1, bufb1, g1)
        return _

    lax.fori_loop(0, NCHC // 2 - 1, pair, None)
    i = NCHC - 2
    wait_gathers(i, bufa0, bufb0, g0)
    add_and_write(i, bufa0, bufb0, wb0, w0)
    wait_gathers(i + 1, bufa1, bufb1, g1)
    add_and_write(i + 1, bufa1, bufb1, wb1, w1)
    wait_write(i, wb0, w0)
    wait_write(i + 1, wb1, w1)


def _tc_two_matmul(x, wa, wb):
    def body(x_ref, wa_ref, wb_ref, oa_ref, ob_ref):
        xv = x_ref[...]
        oa_ref[...] = jnp.dot(xv, wa_ref[...], preferred_element_type=jnp.float32)
        ob_ref[...] = jnp.dot(xv, wb_ref[...], preferred_element_type=jnp.float32)

    return pl.pallas_call(
        body,
        grid=(NPAD // NB,),
        in_specs=[
            pl.BlockSpec((NB, D), lambda i: (i, 0)),
            pl.BlockSpec((D, H), lambda i: (0, 0)),
            pl.BlockSpec((D, H), lambda i: (0, 0)),
        ],
        out_specs=[
            pl.BlockSpec((NB, H), lambda i: (i, 0)),
            pl.BlockSpec((NB, H), lambda i: (i, 0)),
        ],
        out_shape=[
            jax.ShapeDtypeStruct((NPAD, H), jnp.float32),
            jax.ShapeDtypeStruct((NPAD, H), jnp.float32),
        ],
    )(x, wa, wb)


def _tc_sage_update(xs, parts, degp, b, wa, wb, act: bool, pack: bool = False):
    def body(xs_ref, p_ref, dp_ref, b_ref, wa_ref, wb_ref, oa_ref, ob_ref):
        deg = jnp.sum(dp_ref[...], axis=0)
        rdeg = 1.0 / jnp.maximum(deg, 1.0)
        psum = p_ref[0] + p_ref[1]
        h = xs_ref[...] + psum * rdeg[:, None] + b_ref[...]
        if act:
            h = jnp.maximum(h, 0.0)
        ma = jnp.dot(h, wa_ref[...], preferred_element_type=jnp.float32)
        mb = jnp.dot(h, wb_ref[...], preferred_element_type=jnp.float32)
        if pack:
            ua = lax.bitcast_convert_type(ma, jnp.uint32) + jnp.uint32(0x8000)
            ub = lax.bitcast_convert_type(mb, jnp.uint32) + jnp.uint32(0x8000)
            pa = (ua[:, :64] >> 16) | (ua[:, 64:] & jnp.uint32(0xFFFF0000))
            pb = (ub[:, :64] >> 16) | (ub[:, 64:] & jnp.uint32(0xFFFF0000))
            oa_ref[...] = lax.bitcast_convert_type(
                jnp.concatenate([pa[:NB // 2], pa[NB // 2:]], axis=1),
                jnp.int32)
            ob_ref[...] = lax.bitcast_convert_type(
                jnp.concatenate([pb[:NB // 2], pb[NB // 2:]], axis=1),
                jnp.int32)
        else:
            oa_ref[...] = ma
            ob_ref[...] = mb

    return pl.pallas_call(
        body,
        grid=(NPAD // NB,),
        in_specs=[
            pl.BlockSpec((NB, H), lambda i: (i, 0)),
            pl.BlockSpec((NC, NB, H), lambda i: (0, i, 0)),
            pl.BlockSpec((NW, NB), lambda i: (0, i)),
            pl.BlockSpec((1, H), lambda i: (0, 0)),
            pl.BlockSpec((H, H), lambda i: (0, 0)),
            pl.BlockSpec((H, H), lambda i: (0, 0)),
        ],
        out_specs=[
            pl.BlockSpec((NB // 2 if pack else NB, H), lambda i: (i, 0)),
            pl.BlockSpec((NB // 2 if pack else NB, H), lambda i: (i, 0)),
        ],
        out_shape=[
            jax.ShapeDtypeStruct((NPAD // 2 if pack else NPAD, H),
                                 jnp.int32 if pack else jnp.float32),
            jax.ShapeDtypeStruct((NPAD // 2 if pack else NPAD, H),
                                 jnp.int32 if pack else jnp.float32),
        ],
    )(xs, parts, degp, b, wa, wb)


def _tc_edge_mlp(g, efp, wc, bm1, wm2, bm2, wm3, bm3):
    def body(g_ref, ef_ref, wc_ref, b1_ref, w2_ref, b2_ref, w3_ref, b3_ref,
             out_ref):
        gu = lax.bitcast_convert_type(g_ref[...], jnp.uint32)
        outs = []
        for e in range(2):
            ge = gu[:, e * 64:(e + 1) * 64]
            glo = lax.bitcast_convert_type(ge << 16, jnp.float32)
            ghi = lax.bitcast_convert_type(ge & jnp.uint32(0xFFFF0000),
                                           jnp.float32)
            gf = jnp.concatenate([glo, ghi], axis=1)
            efe = ef_ref[:, e * ED:(e + 1) * ED]
            z1 = gf + jnp.dot(efe, wc_ref[...],
                              preferred_element_type=jnp.float32)
            z1 = jnp.maximum(z1 + b1_ref[...], 0.0)
            z2 = jnp.dot(z1, w2_ref[...], preferred_element_type=jnp.float32)
            z2 = jnp.maximum(z2 + b2_ref[...], 0.0)
            outs.append(jnp.dot(z2, w3_ref[...],
                                preferred_element_type=jnp.float32)
                        + b3_ref[...])
        out_ref[...] = jnp.concatenate(outs, axis=1)

    return pl.pallas_call(
        body,
        grid=(E // EB,),
        in_specs=[
            pl.BlockSpec((EB // 2, H), lambda i: (i, 0)),
            pl.BlockSpec((EB // 2, 2 * ED), lambda i: (i, 0)),
            pl.BlockSpec((ED, H), lambda i: (0, 0)),
            pl.BlockSpec((1, H), lambda i: (0, 0)),
            pl.BlockSpec((H, H // 2), lambda i: (0, 0)),
            pl.BlockSpec((1, H // 2), lambda i: (0, 0)),
            pl.BlockSpec((H // 2, 1), lambda i: (0, 0)),
            pl.BlockSpec((1, 1), lambda i: (0, 0)),
        ],
        out_specs=pl.BlockSpec((EB // 2, 2), lambda i: (i, 0)),
        out_shape=jax.ShapeDtypeStruct((E // 2, 2), jnp.float32),
    )(g, efp, wc, bm1, wm2, bm2, wm3, bm3)


def kernel(node_features, edge_index, edge_features,
           W_self1, W_neigh1, b1, W_self2, W_neigh2, b2,
           Wm1, bm1, Wm2, bm2, Wm3, bm3):
    src = edge_index[0]
    dst = edge_index[1]
    dst3d = dst.reshape(NW, NCHA, CHA)
    x = jnp.concatenate(
        [node_features, jnp.zeros((NPAD - N, D), jnp.float32)], axis=0)

    degp = _sc_degree(dst)

    xs1, xw1 = _tc_two_matmul(x, W_self1, W_neigh1)
    p1 = _sc_aggregate(xw1, src, dst3d)
    hs2, hw2 = _tc_sage_update(xs1, p1, degp, b1.reshape(1, H),
                               W_self2, W_neigh2, act=True)

    p2 = _sc_aggregate(hw2, src, dst3d)
    ps, pd = _tc_sage_update(hs2, p2, degp, b2.reshape(1, H),
                             Wm1[:H], Wm1[H:2 * H], act=False, pack=True)

    def _packed_row(n):
        return (n & -1024) | ((n & 511) << 1) | ((n >> 9) & 1)

    g = _sc_edge_combine(ps.reshape(NPAD, H // 2), pd.reshape(NPAD, H // 2),
                         _packed_row(src), _packed_row(dst))
    logits = _tc_edge_mlp(g, edge_features.reshape(E // 2, 2 * ED),
                          Wm1[2 * H:], bm1.reshape(1, H),
                          Wm2, bm2.reshape(1, H // 2), Wm3,
                          bm3.reshape(1, 1))
    return logits.reshape(E)


# speedup vs baseline: 1.1780x; 1.0118x over previous
import functools

import jax
import jax.numpy as jnp
from jax import lax
from jax.experimental import pallas as pl
from jax.experimental.pallas import tpu as pltpu
from jax.experimental.pallas import tpu_sc as plsc

N = 10000
E = 320000
D = 128
ED = 16
H = 128

NPAD = 10240
NB = 1024
EB = 6400

NC = 2
NS = 16
NW = NC * NS
EPW = E // NW
CHA = 80
NCHA = EPW // CHA
CHC = 200
NCHC = EPW // CHC
RPT = NPAD // NS
_MESH = plsc.VectorSubcoreMesh(core_axis_name="c", subcore_axis_name="s")


def _zero_rows(buf, rows, width):
    zero16 = jnp.zeros((16,), jnp.float32)

    def body(i, _):
        for j in range(width // 16):
            buf[i, pl.ds(j * 16, 16)] = zero16
        return _

    lax.fori_loop(0, rows, body, None)


_SC_PARAMS = pltpu.CompilerParams(needs_layout_passes=False)


HALF = 64


def _build_sc_aggregate(with_deg: bool):
    nidx = HALF if with_deg else NCHA
    out_type = [jax.ShapeDtypeStruct((NC, NPAD, H), jnp.float32)]
    scratch = [
        pltpu.VMEM((nidx * CHA,), jnp.int32),
        pltpu.VMEM((nidx, CHA), jnp.int32),
        pltpu.VMEM((CHA, H), jnp.float32),
        pltpu.VMEM((CHA, H), jnp.float32),
        pltpu.VMEM_SHARED((NPAD, H), jnp.float32),
        pltpu.SemaphoreType.DMA,
        pltpu.SemaphoreType.DMA,
        pltpu.SemaphoreType.DMA,
        pltpu.SemaphoreType.DMA,
    ]
    if with_deg:
        out_type.append(jax.ShapeDtypeStruct((NW, NPAD), jnp.float32))
        scratch.insert(4, pltpu.VMEM((NPAD,), jnp.float32))

    def body(table, src_hbm, dst3d_hbm, *refs):
        if with_deg:
            (out_agg, out_deg, src_all, dst_all, rows0, rows1, deg_v, acc,
             g0, g1, s0, s1) = refs
        else:
            (out_agg, src_all, dst_all, rows0, rows1, acc, g0, g1, s0,
             s1) = refs
        c = lax.axis_index("c")
        s = lax.axis_index("s")
        wid = c * NS + s
        ones16 = jnp.ones((16,), jnp.float32)
        zero16 = jnp.zeros((16,), jnp.float32)

        pltpu.sync_copy(src_hbm.at[pl.ds(wid * EPW, nidx * CHA)], src_all)
        if with_deg:
            pltpu.sync_copy(dst3d_hbm.at[wid, pl.ds(0, HALF)], dst_all)
        else:
            pltpu.sync_copy(dst3d_hbm.at[wid], dst_all)

        _zero_rows(rows0, CHA, H)
        for k in range(RPT // CHA):
            pltpu.sync_copy(rows0, acc.at[pl.ds(s * RPT + k * CHA, CHA)])
        if with_deg:
            def zdeg(i, _):
                deg_v[pl.ds(i * 16, 16)] = zero16
                return _

            lax.fori_loop(0, NPAD // 16, zdeg, None)
        plsc.subcore_barrier()

        def gather(i, boff, rows_v, sem):
            pltpu.async_copy(
                table.at[src_all.at[pl.ds((i - boff) * CHA, CHA)]],
                rows_v, sem)

        def wait_gather(i, boff, rows_v, sem):
            pltpu.make_async_copy(
                table.at[src_all.at[pl.ds((i - boff) * CHA, CHA)]],
                rows_v, sem).wait()

        def scatter(i, boff, rows_v, sem):
            pltpu.async_copy(rows_v, acc.at[dst_all.at[i - boff]], sem,
                             add=True)
            if with_deg:
                for j in range(CHA // 16):
                    idx = dst_all[i - boff, pl.ds(j * 16, 16)]
                    plsc.addupdate_scatter(deg_v, [idx], ones16)

        def wait_scatter(i, boff, rows_v, sem):
            pltpu.make_async_copy(rows_v, acc.at[dst_all.at[i - boff]],
                                  sem).wait()

        def make_pair(boff):
            def pair(k, _):
                i = 2 * k
                wait_gather(i, boff, rows0, g0)
                scatter(i, boff, rows0, s0)
                wait_gather(i + 1, boff, rows1, g1)
                scatter(i + 1, boff, rows1, s1)
                wait_scatter(i, boff, rows0, s0)
                gather(i + 2, boff, rows0, g0)
                wait_scatter(i + 1, boff, rows1, s1)
                gather(i + 3, boff, rows1, g1)
                return _

            return pair

        gather(0, 0, rows0, g0)
        gather(1, 0, rows1, g1)

        if with_deg:
            lax.fori_loop(0, (HALF - 2) // 2, make_pair(0), None)
            i = HALF - 2
            wait_gather(i, 0, rows0, g0)
            scatter(i, 0, rows0, s0)
            wait_gather(i + 1, 0, rows1, g1)
            scatter(i + 1, 0, rows1, s1)
            pltpu.sync_copy(
                src_hbm.at[pl.ds(wid * EPW + HALF * CHA,
                                 (NCHA - HALF) * CHA)],
                src_all.at[pl.ds(0, (NCHA - HALF) * CHA)])
            pltpu.sync_copy(dst3d_hbm.at[wid, pl.ds(HALF, NCHA - HALF)],
                            dst_all.at[pl.ds(0, NCHA - HALF)])
            wait_scatter(i, 0, rows0, s0)
            gather(i + 2, HALF, rows0, g0)
            wait_scatter(i + 1, 0, rows1, s1)
            gather(i + 3, HALF, rows1, g1)
            base_pair = HALF // 2
            lax.fori_loop(base_pair, (NCHA - 3) // 2, make_pair(HALF), None)
            boff = HALF
        else:
            lax.fori_loop(0, (NCHA - 3) // 2, make_pair(0), None)
            boff = 0

        i = NCHA - 3
        wait_gather(i, boff, rows0, g0)
        scatter(i, boff, rows0, s0)
        wait_gather(i + 1, boff, rows1, g1)
        scatter(i + 1, boff, rows1, s1)
        wait_scatter(i, boff, rows0, s0)
        gather(i + 2, boff, rows0, g0)
        wait_gather(i + 2, boff, rows0, g0)
        scatter(i + 2, boff, rows0, s0)
        wait_scatter(i + 1, boff, rows1, s1)
        wait_scatter(i + 2, boff, rows0, s0)
        plsc.subcore_barrier()

        r0 = s * RPT
        for k in range(RPT // CHA):
            pltpu.sync_copy(acc.at[pl.ds(r0 + k * CHA, CHA)], rows0)
            pltpu.sync_copy(rows0, out_agg.at[c, pl.ds(r0 + k * CHA, CHA)])
        if with_deg:
            pltpu.sync_copy(deg_v, out_deg.at[wid])

    return pl.kernel(body, out_type=tuple(out_type), mesh=_MESH,
                     scratch_types=scratch, compiler_params=_SC_PARAMS)


_sc_aggregate_deg = _build_sc_aggregate(True)
_sc_aggregate = _build_sc_aggregate(False)


@functools.partial(
    pl.kernel,
    out_type=jax.ShapeDtypeStruct((E // 2, H), jnp.int32),
    mesh=_MESH,
    scratch_types=[
        pltpu.VMEM((EPW,), jnp.int32),
        pltpu.VMEM((EPW,), jnp.int32),
        pltpu.VMEM((CHC, H // 2), jnp.int32),
        pltpu.VMEM((CHC, H // 2), jnp.int32),
        pltpu.VMEM((CHC, H // 2), jnp.int32),
        pltpu.VMEM((CHC, H // 2), jnp.int32),
        pltpu.VMEM((CHC // 2, H), jnp.int32),
        pltpu.VMEM((CHC // 2, H), jnp.int32),
        pltpu.SemaphoreType.DMA,
        pltpu.SemaphoreType.DMA,
        pltpu.SemaphoreType.DMA,
        pltpu.SemaphoreType.DMA,
    ],
    compiler_params=pltpu.CompilerParams(needs_layout_passes=False,
                                         use_tc_tiling_on_sc=False),
)
def _sc_edge_combine(ps_hbm, pd_hbm, src_hbm, dst_hbm, out_g,
                     src_all, dst_all, bufa0, bufa1, bufb0, bufb1,
                     wb0, wb1, g0, g1, w0, w1):
    c = lax.axis_index("c")
    s = lax.axis_index("s")
    wid = c * NS + s
    base = wid * EPW
    pltpu.sync_copy(src_hbm.at[pl.ds(base, EPW)], src_all)
    pltpu.sync_copy(dst_hbm.at[pl.ds(base, EPW)], dst_all)

    def gathers(i, buf_a, buf_b, sem):
        sl = pl.ds(i * CHC, CHC)
        pltpu.async_copy(ps_hbm.at[src_all.at[sl]], buf_a, sem)
        pltpu.async_copy(pd_hbm.at[dst_all.at[sl]], buf_b, sem)

    def wait_gathers(i, buf_a, buf_b, sem):
        sl = pl.ds(i * CHC, CHC)
        pltpu.make_async_copy(ps_hbm.at[src_all.at[sl]], buf_a, sem).wait()
        pltpu.make_async_copy(pd_hbm.at[dst_all.at[sl]], buf_b, sem).wait()

    def add_and_write(i, buf_a, buf_b, wbuf, sem):
        def add_row(r2, _):
            for e in range(2):
                for j in range(H // 32):
                    sl = pl.ds(j * 16, 16)
                    a = plsc.bitcast(buf_a[2 * r2 + e, sl], jnp.bfloat16)
                    bv = plsc.bitcast(buf_b[2 * r2 + e, sl], jnp.bfloat16)
                    wbuf[r2, pl.ds(e * 64 + j * 16, 16)] = plsc.bitcast(
                        a + bv, jnp.int32)
            return _

        lax.fori_loop(0, CHC // 2, add_row, None)
        pltpu.async_copy(
            wbuf, out_g.at[pl.ds((base + i * CHC) // 2, CHC // 2)], sem)

    def wait_write(i, wbuf, sem):
        pltpu.make_async_copy(
            wbuf, out_g.at[pl.ds((base + i * CHC) // 2, CHC // 2)],
            sem).wait()

    gathers(0, bufa0, bufb0, g0)
    gathers(1, bufa1, bufb1, g1)

    def pair(k, _):
        i = 2 * k
        wait_gathers(i, bufa0, bufb0, g0)
        add_and_write(i, bufa0, bufb0, wb0, w0)
        wait_gathers(i + 1, bufa1, bufb1, g1)
        add_and_write(i + 1, bufa1, bufb1, wb1, w1)
        wait_write(i, wb0, w0)
        gathers(i + 2, bufa0, bufb0, g0)
        wait_write(i + 1, wb1, w1)
        gathers(i + 3, bufa1, bufb1, g1)
        return _

    lax.fori_loop(0, NCHC // 2 - 1, pair, None)
    i = NCHC - 2
    wait_gathers(i, bufa0, bufb0, g0)
    add_and_write(i, bufa0, bufb0, wb0, w0)
    wait_gathers(i + 1, bufa1, bufb1, g1)
    add_and_write(i + 1, bufa1, bufb1, wb1, w1)
    wait_write(i, wb0, w0)
    wait_write(i + 1, wb1, w1)


def _tc_two_matmul(x, wa, wb):
    def body(x_ref, wa_ref, wb_ref, oa_ref, ob_ref):
        xv = x_ref[...]
        oa_ref[...] = jnp.dot(xv, wa_ref[...], preferred_element_type=jnp.float32)
        ob_ref[...] = jnp.dot(xv, wb_ref[...], preferred_element_type=jnp.float32)

    return pl.pallas_call(
        body,
        grid=(NPAD // NB,),
        in_specs=[
            pl.BlockSpec((NB, D), lambda i: (i, 0)),
            pl.BlockSpec((D, H), lambda i: (0, 0)),
            pl.BlockSpec((D, H), lambda i: (0, 0)),
        ],
        out_specs=[
            pl.BlockSpec((NB, H), lambda i: (i, 0)),
            pl.BlockSpec((NB, H), lambda i: (i, 0)),
        ],
        out_shape=[
            jax.ShapeDtypeStruct((NPAD, H), jnp.float32),
            jax.ShapeDtypeStruct((NPAD, H), jnp.float32),
        ],
    )(x, wa, wb)


def _tc_sage_update(xs, parts, degp, b, wa, wb, act: bool, pack: bool = False):
    def body(xs_ref, p_ref, dp_ref, b_ref, wa_ref, wb_ref, oa_ref, ob_ref):
        deg = jnp.sum(dp_ref[...], axis=0)
        rdeg = 1.0 / jnp.maximum(deg, 1.0)
        psum = p_ref[0] + p_ref[1]
        h = xs_ref[...] + psum * rdeg[:, None] + b_ref[...]
        if act:
            h = jnp.maximum(h, 0.0)
        ma = jnp.dot(h, wa_ref[...], preferred_element_type=jnp.float32)
        mb = jnp.dot(h, wb_ref[...], preferred_element_type=jnp.float32)
        if pack:
            ua = lax.bitcast_convert_type(ma, jnp.uint32) + jnp.uint32(0x8000)
            ub = lax.bitcast_convert_type(mb, jnp.uint32) + jnp.uint32(0x8000)
            pa = (ua[:, :64] >> 16) | (ua[:, 64:] & jnp.uint32(0xFFFF0000))
            pb = (ub[:, :64] >> 16) | (ub[:, 64:] & jnp.uint32(0xFFFF0000))
            oa_ref[...] = lax.bitcast_convert_type(
                jnp.concatenate([pa[:NB // 2], pa[NB // 2:]], axis=1),
                jnp.int32)
            ob_ref[...] = lax.bitcast_convert_type(
                jnp.concatenate([pb[:NB // 2], pb[NB // 2:]], axis=1),
                jnp.int32)
        else:
            oa_ref[...] = ma
            ob_ref[...] = mb

    return pl.pallas_call(
        body,
        grid=(NPAD // NB,),
        in_specs=[
            pl.BlockSpec((NB, H), lambda i: (i, 0)),
            pl.BlockSpec((NC, NB, H), lambda i: (0, i, 0)),
            pl.BlockSpec((NW, NB), lambda i: (0, i)),
            pl.BlockSpec((1, H), lambda i: (0, 0)),
            pl.BlockSpec((H, H), lambda i: (0, 0)),
            pl.BlockSpec((H, H), lambda i: (0, 0)),
        ],
        out_specs=[
            pl.BlockSpec((NB // 2 if pack else NB, H), lambda i: (i, 0)),
            pl.BlockSpec((NB // 2 if pack else NB, H), lambda i: (i, 0)),
        ],
        out_shape=[
            jax.ShapeDtypeStruct((NPAD // 2 if pack else NPAD, H),
                                 jnp.int32 if pack else jnp.float32),
            jax.ShapeDtypeStruct((NPAD // 2 if pack else NPAD, H),
                                 jnp.int32 if pack else jnp.float32),
        ],
    )(xs, parts, degp, b, wa, wb)


def _tc_edge_mlp(g, efp, wc, bm1, wm2, bm2, wm3, bm3):
    def body(g_ref, ef_ref, wc_ref, b1_ref, w2_ref, b2_ref, w3_ref, b3_ref,
             out_ref):
        gu = lax.bitcast_convert_type(g_ref[...], jnp.uint32)
        outs = []
        for e in range(2):
            ge = gu[:, e * 64:(e + 1) * 64]
            glo = lax.bitcast_convert_type(ge << 16, jnp.float32)
            ghi = lax.bitcast_convert_type(ge & jnp.uint32(0xFFFF0000),
                                           jnp.float32)
            gf = jnp.concatenate([glo, ghi], axis=1)
            efe = ef_ref[:, e * ED:(e + 1) * ED]
            z1 = gf + jnp.dot(efe, wc_ref[...],
                              preferred_element_type=jnp.float32)
            z1 = jnp.maximum(z1 + b1_ref[...], 0.0)
            z2 = jnp.dot(z1, w2_ref[...], preferred_element_type=jnp.float32)
            z2 = jnp.maximum(z2 + b2_ref[...], 0.0)
            outs.append(jnp.dot(z2, w3_ref[...],
                                preferred_element_type=jnp.float32)
                        + b3_ref[...])
        out_ref[...] = jnp.concatenate(outs, axis=1)

    return pl.pallas_call(
        body,
        grid=(E // EB,),
        in_specs=[
            pl.BlockSpec((EB // 2, H), lambda i: (i, 0)),
            pl.BlockSpec((EB // 2, 2 * ED), lambda i: (i, 0)),
            pl.BlockSpec((ED, H), lambda i: (0, 0)),
            pl.BlockSpec((1, H), lambda i: (0, 0)),
            pl.BlockSpec((H, H // 2), lambda i: (0, 0)),
            pl.BlockSpec((1, H // 2), lambda i: (0, 0)),
            pl.BlockSpec((H // 2, 1), lambda i: (0, 0)),
            pl.BlockSpec((1, 1), lambda i: (0, 0)),
        ],
        out_specs=pl.BlockSpec((EB // 2, 2), lambda i: (i, 0)),
        out_shape=jax.ShapeDtypeStruct((E // 2, 2), jnp.float32),
    )(g, efp, wc, bm1, wm2, bm2, wm3, bm3)


def kernel(node_features, edge_index, edge_features,
           W_self1, W_neigh1, b1, W_self2, W_neigh2, b2,
           Wm1, bm1, Wm2, bm2, Wm3, bm3):
    src = edge_index[0]
    dst = edge_index[1]
    dst3d = dst.reshape(NW, NCHA, CHA)
    x = jnp.concatenate(
        [node_features, jnp.zeros((NPAD - N, D), jnp.float32)], axis=0)

    xs1, xw1 = _tc_two_matmul(x, W_self1, W_neigh1)
    p1, degp = _sc_aggregate_deg(xw1, src, dst3d)
    hs2, hw2 = _tc_sage_update(xs1, p1, degp, b1.reshape(1, H),
                               W_self2, W_neigh2, act=True)

    (p2,) = _sc_aggregate(hw2, src, dst3d)
    ps, pd = _tc_sage_update(hs2, p2, degp, b2.reshape(1, H),
                             Wm1[:H], Wm1[H:2 * H], act=False, pack=True)

    def _packed_row(n):
        return (n & -1024) | ((n & 511) << 1) | ((n >> 9) & 1)

    g = _sc_edge_combine(ps.reshape(NPAD, H // 2), pd.reshape(NPAD, H // 2),
                         _packed_row(src), _packed_row(dst))
    logits = _tc_edge_mlp(g, edge_features.reshape(E // 2, 2 * ED),
                          Wm1[2 * H:], bm1.reshape(1, H),
                          Wm2, bm2.reshape(1, H // 2), Wm3,
                          bm3.reshape(1, 1))
    return logits.reshape(E)
